# baseline (speedup 1.0000x reference)
import jax
import jax.numpy as jnp
import numpy as np
from jax import lax
from jax.experimental import pallas as pl
from jax.experimental.pallas import tpu as pltpu
from jax.experimental.pallas import tpu_sc as plsc

_B = 20
_NPG = 500
_N = 10000
_E = 320000
_D = 128
_FF = 256
_L = 3
_KK = 16
_C = 10

_NC = 2
_NS = 16
_EPG = _E // _B
_GPC = _B // _NC
_WAVES = 5
_GPW = _GPC // _WAVES
_RT = (_NPG + 7) // 8
_CT = (_NPG + 127) // 128
_TCELLS = _RT * _CT * 1024
_WAVE_CELLS = _GPW * _TCELLS
_WAVE_EDGES = _GPW * _EPG
_EPT = _WAVE_EDGES // _NS
_CH = 128
_NCHUNK = (_EPT + _CH - 1) // _CH
_EBUF = _NCHUNK * _CH
_SLICE = _WAVE_CELLS // _NS
_SPMEM = _WAVE_CELLS + 8
_DUMP = _WAVE_CELLS


def _adj_body(src_hbm, dst_hbm, a_hbm, zero_v, stage_v, src_v, dst_v, idx_v,
              ones_v, shared):
    cid = lax.axis_index("c")
    sid = lax.axis_index("s")

    zf = jnp.zeros((16,), jnp.float32)
    of = jnp.ones((16,), jnp.float32)

    def fill_body(j, _):
        zero_v[pl.ds(j * 16, 16)] = zf
        return 0

    lax.fori_loop(0, _SLICE // 16, fill_body, 0)
    for j in range(_CH // 16):
        ones_v[pl.ds(j * 16, 16)] = of

    for w in range(_WAVES):
        g0 = cid * _GPC + w * _GPW
        pltpu.sync_copy(zero_v, shared.at[pl.ds(sid * _SLICE, _SLICE)])
        plsc.subcore_barrier()
        ebase = g0 * _EPG + sid * _EPT
        pltpu.sync_copy(src_hbm.at[pl.ds(ebase, _EPT)],
                        src_v.at[pl.ds(0, _EPT)])
        pltpu.sync_copy(dst_hbm.at[pl.ds(ebase, _EPT)],
                        dst_v.at[pl.ds(0, _EPT)])
        lanes = lax.broadcasted_iota(jnp.int32, (16,), 0)

        def chunk_body(c, _):
            for j in range(_CH // 16):
                off = c * _CH + j * 16
                s = src_v[pl.ds(off, 16)]
                dv = dst_v[pl.ds(off, 16)]
                g = lax.shift_right_logical(dv * 16778, 23)
                dl = dv - g * _NPG
                sl = s - g * _NPG
                widx = ((g - g0) * _TCELLS
                        + (lax.shift_right_logical(dl, 3) * _CT
                           + lax.shift_right_logical(sl, 7)) * 1024
                        + (dl & 7) * 128 + (sl & 127))
                valid = (off + lanes) < _EPT
                idx_v[pl.ds(j * 16, 16)] = jnp.where(valid, widx, _DUMP)
            pltpu.sync_copy(ones_v, shared.at[idx_v], add=True)
            return 0

        lax.fori_loop(0, _NCHUNK, chunk_body, 0)
        plsc.subcore_barrier()
        awave = g0 * _TCELLS
        pltpu.sync_copy(shared.at[pl.ds(sid * _SLICE, _SLICE)], stage_v)
        pltpu.sync_copy(stage_v,
                        a_hbm.at[pl.ds(awave + sid * _SLICE, _SLICE)])
        plsc.subcore_barrier()


def _build_adjacency(edge_index):
    mesh = plsc.VectorSubcoreMesh(core_axis_name="c", subcore_axis_name="s",
                                  num_cores=_NC, num_subcores=_NS)
    fn = pl.kernel(
        _adj_body,
        out_type=jax.ShapeDtypeStruct((_B * _TCELLS,), jnp.float32),
        mesh=mesh,
        scratch_types=[
            pltpu.VMEM((_SLICE,), jnp.float32),
            pltpu.VMEM((_SLICE,), jnp.float32),
            pltpu.VMEM((_EBUF,), jnp.int32),
            pltpu.VMEM((_EBUF,), jnp.int32),
            pltpu.VMEM((_CH,), jnp.int32),
            pltpu.VMEM((_CH,), jnp.float32),
            pltpu.VMEM_SHARED((_SPMEM,), jnp.float32),
        ],
    )
    return fn(edge_index[0], edge_index[1])


def _mm(a, b):
    return jnp.dot(a, b, preferred_element_type=jnp.float32)


def _attn_core(z, xd, a_blk, st, bns, bnsh, wq, wk, wg, bg):
    mu = st[0:1] * (1.0 / _N)
    ex2 = st[1:2] * (1.0 / _N)
    var = ex2 - mu * mu
    xu = (z - mu) / jnp.sqrt(var + 1e-5) * bns + bnsh
    q = _mm(xu, wq)
    k = _mm(xu, wk)
    logits = lax.dot_general(q, k, (((1,), (1,)), ((), ())),
                             preferred_element_type=jnp.float32)
    logits = logits * (1.0 / np.sqrt(_D))
    neg = jnp.full((_NPG, 12), -3e38, jnp.float32)
    x0 = logits[:, 0:128]
    x1 = logits[:, 128:256]
    x2 = logits[:, 256:384]
    x3 = jnp.concatenate([logits[:, 384:500], neg], axis=1)
    a = jnp.maximum(x0, x1)
    b = jnp.minimum(x0, x1)
    c = jnp.maximum(x2, x3)
    d = jnp.minimum(x2, x3)
    s0 = jnp.maximum(a, c)
    t = jnp.minimum(a, c)
    u = jnp.maximum(b, d)
    n3 = jnp.minimum(b, d)
    n1 = jnp.maximum(t, u)
    n2 = jnp.minimum(t, u)
    work = s0
    mx = jnp.max(work, axis=-1, keepdims=True)
    m = mx
    for _t in range(_KK - 1):
        eq = work == mx
        work = jnp.where(eq, n1, work)
        n1 = jnp.where(eq, n2, n1)
        n2 = jnp.where(eq, n3, n2)
        n3 = jnp.where(eq, -3e38, n3)
        mx = jnp.max(work, axis=-1, keepdims=True)
    e = jnp.exp(logits - m)
    zs = jnp.sum(e, axis=-1, keepdims=True)
    sel = logits >= mx
    esel = jnp.where(sel, e, 0.0)
    denom = jnp.sum(esel, axis=-1, keepdims=True) + zs * 1e-9
    wfull = esel / denom
    agg = _mm(wfull, xd)
    xpad = jnp.concatenate(
        [xd[3 * 128:], jnp.zeros((4 * 128 - _NPG, _D), jnp.float32)], axis=0)
    seg_p = jnp.zeros((_RT * 8, _D), jnp.float32)
    deg_p = jnp.zeros((_RT * 8, 1), jnp.float32)
    for cg in range(_CT):
        acg = a_blk[:, cg].reshape(_RT * 8, 128)
        xr = xd[cg * 128:(cg + 1) * 128] if cg < _CT - 1 else xpad
        seg_p = seg_p + _mm(acg, xr)
        deg_p = deg_p + jnp.sum(acg, axis=-1, keepdims=True)
    seg = seg_p[:_NPG] / jnp.maximum(deg_p[:_NPG], 1.0)
    pre = _mm(seg, wg[:_D]) + _mm(agg, wg[_D:]) + bg
    return xu, xd + jnp.maximum(pre, 0.0)


def _ffn(xin, w1, b1, w2, b2):
    return xin + _mm(jnp.maximum(_mm(xin, w1) + b1, 0.0), w2) + b2


def _fused_body(x_ref, a_ref, wenc_ref, benc_ref, w1_ref, b1_ref, w2_ref,
                b2_ref, bns_ref, bnsh_ref, wq_ref, wk_ref, wg_ref, bg_ref,
                wm1_ref, bm1_ref, wm2_ref, bm2_ref, out_ref,
                a_s, z_s, xd_s, st_s, pool_s):
    i = pl.program_id(0)
    b = pl.program_id(1)

    def put_stats(slot, znew):
        contrib = jnp.concatenate(
            [jnp.sum(znew, axis=0, keepdims=True),
             jnp.sum(znew * znew, axis=0, keepdims=True)], axis=0)
        prev = jnp.where(b == 0, 0.0, st_s[slot][0])
        st_s[slot] = (prev + contrib)[None]

    @pl.when(i == 0)
    def _():
        a_s[pl.ds(b, 1)] = a_ref[...]
        h = _mm(x_ref[0], wenc_ref[...]) + benc_ref[...]
        z1 = _ffn(h, w1_ref[0], b1_ref[0], w2_ref[0], b2_ref[0])
        xd_s[pl.ds(b, 1)] = h[None]
        z_s[pl.ds(b, 1)] = z1[None]
        put_stats(pl.ds(0, 1), z1)

    @pl.when(i > 0)
    def _():
        zb = z_s[pl.ds(b, 1)][0]
        xdb = xd_s[pl.ds(b, 1)][0]
        ab = a_s[pl.ds(b, 1)][0]
        st = st_s[pl.ds(i - 1, 1)][0]
        xu, xdn = _attn_core(zb, xdb, ab, st, bns_ref[0], bnsh_ref[0],
                             wq_ref[0], wk_ref[0], wg_ref[0], bg_ref[0])
        xd_s[pl.ds(b, 1)] = xdn[None]

        @pl.when(i < _L)
        def _():
            z2 = _ffn(xu, w1_ref[0], b1_ref[0], w2_ref[0], b2_ref[0])
            z_s[pl.ds(b, 1)] = z2[None]
            put_stats(pl.ds(i, 1), z2)

        @pl.when(i == _L)
        def _():
            pool_s[pl.ds(b, 1)] = jnp.sum(
                xdn, axis=0, keepdims=True) * (1.0 / _NPG)

            @pl.when(b == _B - 1)
            def _():
                pooled = pool_s[...][:_B]
                hmid = jnp.maximum(_mm(pooled, wm1_ref[...]) + bm1_ref[...],
                                   0.0)
                out_ref[...] = _mm(hmid, wm2_ref[...]) + bm2_ref[...]


def _c0(shape):
    return pl.BlockSpec(shape, lambda i, b: tuple(0 for _ in shape))


def kernel(x, edge_index, batch, W_enc, b_enc, W1, b1, W2, b2, bn_scale,
           bn_shift, Wq, Wk, W_gnn, b_gnn, Wm1, bm1, Wm2, bm2):
    a5 = _build_adjacency(edge_index).reshape(_B, _RT, _CT, 8, 128)
    f32 = jnp.float32

    def enc_b(i, b):
        return (jnp.where(i == 0, b, 0), 0, 0)

    def enc_b5(i, b):
        return (jnp.where(i == 0, b, 0), 0, 0, 0, 0)

    def ffn_i(i, b):
        return (jnp.minimum(i, _L - 1), 0, 0)

    def lay_i(i, b):
        return (jnp.maximum(i - 1, 0), 0, 0)

    out = pl.pallas_call(
        _fused_body,
        grid=(_L + 1, _B),
        in_specs=[
            pl.BlockSpec((1, _NPG, _D), enc_b),
            pl.BlockSpec((1, _RT, _CT, 8, 128), enc_b5),
            _c0((_D, _D)),
            _c0((1, _D)),
            pl.BlockSpec((1, _D, _FF), ffn_i),
            pl.BlockSpec((1, 1, _FF), ffn_i),
            pl.BlockSpec((1, _FF, _D), ffn_i),
            pl.BlockSpec((1, 1, _D), ffn_i),
            pl.BlockSpec((1, 1, _D), lay_i),
            pl.BlockSpec((1, 1, _D), lay_i),
            pl.BlockSpec((1, _D, _D), lay_i),
            pl.BlockSpec((1, _D, _D), lay_i),
            pl.BlockSpec((1, 2 * _D, _D), lay_i),
            pl.BlockSpec((1, 1, _D), lay_i),
            _c0((_D, _D)),
            _c0((1, _D)),
            _c0((_D, _C)),
            _c0((1, _C)),
        ],
        out_specs=pl.BlockSpec((_B, _C), lambda i, b: (0, 0)),
        out_shape=jax.ShapeDtypeStruct((_B, _C), f32),
        compiler_params=pltpu.CompilerParams(
            vmem_limit_bytes=56 * 1024 * 1024),
        scratch_shapes=[
            pltpu.VMEM((_B, _RT, _CT, 8, 128), f32),
            pltpu.VMEM((_B, _NPG, _D), f32),
            pltpu.VMEM((_B, _NPG, _D), f32),
            pltpu.VMEM((_L, 2, _D), f32),
            pltpu.VMEM((24, _D), f32),
        ],
    )(x.reshape(_B, _NPG, _D), a5, W_enc, b_enc.reshape(1, _D), W1,
      b1.reshape(_L, 1, _FF), W2, b2.reshape(_L, 1, _D),
      bn_scale.reshape(_L, 1, _D), bn_shift.reshape(_L, 1, _D), Wq, Wk,
      W_gnn, b_gnn.reshape(_L, 1, _D), Wm1, bm1.reshape(1, _D), Wm2,
      bm2.reshape(1, _C))
    return out


# speedup vs baseline: 1.0752x; 1.0752x over previous
import jax
import jax.numpy as jnp
import numpy as np
from jax import lax
from jax.experimental import pallas as pl
from jax.experimental.pallas import tpu as pltpu
from jax.experimental.pallas import tpu_sc as plsc

_B = 20
_NPG = 500
_N = 10000
_E = 320000
_D = 128
_FF = 256
_L = 3
_KK = 16
_C = 10

_NC = 2
_NS = 16
_EPG = _E // _B
_GPC = _B // _NC
_WAVES = 5
_GPW = _GPC // _WAVES
_RT = (_NPG + 7) // 8
_CT = (_NPG + 127) // 128
_TCELLS = _RT * _CT * 1024
_WAVE_CELLS = _GPW * _TCELLS
_WAVE_EDGES = _GPW * _EPG
_EPT = _WAVE_EDGES // _NS
_CH = 128
_NCHUNK = (_EPT + _CH - 1) // _CH
_EBUF = _NCHUNK * _CH
_SLICE = _WAVE_CELLS // _NS
_SPMEM = _WAVE_CELLS + 8
_DUMP = _WAVE_CELLS


def _adj_body(src_hbm, dst_hbm, a_hbm, zero_v, stage_v, src_v, dst_v, idx_v,
              ones_v, shared):
    cid = lax.axis_index("c")
    sid = lax.axis_index("s")

    zf = jnp.zeros((16,), jnp.float32)
    of = jnp.ones((16,), jnp.float32)

    def fill_body(j, _):
        zero_v[pl.ds(j * 16, 16)] = zf
        return 0

    lax.fori_loop(0, _SLICE // 16, fill_body, 0)
    for j in range(_CH // 16):
        ones_v[pl.ds(j * 16, 16)] = of

    for w in range(_WAVES):
        g0 = cid * _GPC + w * _GPW
        pltpu.sync_copy(zero_v, shared.at[pl.ds(sid * _SLICE, _SLICE)])
        plsc.subcore_barrier()
        ebase = g0 * _EPG + sid * _EPT
        pltpu.sync_copy(src_hbm.at[pl.ds(ebase, _EPT)],
                        src_v.at[pl.ds(0, _EPT)])
        pltpu.sync_copy(dst_hbm.at[pl.ds(ebase, _EPT)],
                        dst_v.at[pl.ds(0, _EPT)])
        lanes = lax.broadcasted_iota(jnp.int32, (16,), 0)

        def chunk_body(c, _):
            for j in range(_CH // 16):
                off = c * _CH + j * 16
                s = src_v[pl.ds(off, 16)]
                dv = dst_v[pl.ds(off, 16)]
                g = lax.shift_right_logical(dv * 16778, 23)
                dl = dv - g * _NPG
                sl = s - g * _NPG
                widx = ((g - g0) * _TCELLS
                        + (lax.shift_right_logical(dl, 3) * _CT
                           + lax.shift_right_logical(sl, 7)) * 1024
                        + (dl & 7) * 128 + (sl & 127))
                valid = (off + lanes) < _EPT
                idx_v[pl.ds(j * 16, 16)] = jnp.where(valid, widx, _DUMP)
            pltpu.sync_copy(ones_v, shared.at[idx_v], add=True)
            return 0

        lax.fori_loop(0, _NCHUNK, chunk_body, 0)
        plsc.subcore_barrier()
        awave = g0 * _TCELLS
        pltpu.sync_copy(shared.at[pl.ds(sid * _SLICE, _SLICE)], stage_v)
        pltpu.sync_copy(stage_v,
                        a_hbm.at[pl.ds(awave + sid * _SLICE, _SLICE)])
        plsc.subcore_barrier()


def _build_adjacency(edge_index):
    mesh = plsc.VectorSubcoreMesh(core_axis_name="c", subcore_axis_name="s",
                                  num_cores=_NC, num_subcores=_NS)
    fn = pl.kernel(
        _adj_body,
        out_type=jax.ShapeDtypeStruct((_B * _TCELLS,), jnp.float32),
        mesh=mesh,
        scratch_types=[
            pltpu.VMEM((_SLICE,), jnp.float32),
            pltpu.VMEM((_SLICE,), jnp.float32),
            pltpu.VMEM((_EBUF,), jnp.int32),
            pltpu.VMEM((_EBUF,), jnp.int32),
            pltpu.VMEM((_CH,), jnp.int32),
            pltpu.VMEM((_CH,), jnp.float32),
            pltpu.VMEM_SHARED((_SPMEM,), jnp.float32),
        ],
    )
    return fn(edge_index[0], edge_index[1])


def _mm(a, b):
    return jnp.dot(a, b, preferred_element_type=jnp.float32)


def _enc_body(x_ref, wenc_ref, benc_ref, w1_ref, b1_ref, w2_ref, b2_ref,
              h_ref, z_ref, ssum_ref, ssq_ref):
    xg = x_ref[0]
    h = _mm(xg, wenc_ref[...]) + benc_ref[...]
    z = h + _mm(jnp.maximum(_mm(h, w1_ref[...]) + b1_ref[...], 0.0),
                w2_ref[...]) + b2_ref[...]
    h_ref[0] = h
    z_ref[0] = z
    ssum_ref[0] = jnp.sum(z, axis=0, keepdims=True)
    ssq_ref[0] = jnp.sum(z * z, axis=0, keepdims=True)


def _attn_update(z, xd, a_blk, ssum, ssq, bns, bnsh, wq, wk, wg, bg):
    mu = jnp.sum(ssum, axis=0) * (1.0 / _N)
    ex2 = jnp.sum(ssq, axis=0) * (1.0 / _N)
    var = ex2 - mu * mu
    xu = (z - mu) / jnp.sqrt(var + 1e-5) * bns + bnsh
    q = _mm(xu, wq)
    k = _mm(xu, wk)
    logits = lax.dot_general(q, k, (((1,), (1,)), ((), ())),
                             preferred_element_type=jnp.float32)
    logits = logits * (1.0 / np.sqrt(_D))
    neg = jnp.full((_NPG, 12), -3e38, jnp.float32)
    x0 = logits[:, 0:128]
    x1 = logits[:, 128:256]
    x2 = logits[:, 256:384]
    x3 = jnp.concatenate([logits[:, 384:500], neg], axis=1)
    a = jnp.maximum(x0, x1)
    b = jnp.minimum(x0, x1)
    c = jnp.maximum(x2, x3)
    d = jnp.minimum(x2, x3)
    s0 = jnp.maximum(a, c)
    t = jnp.minimum(a, c)
    u = jnp.maximum(b, d)
    n3 = jnp.minimum(b, d)
    n1 = jnp.maximum(t, u)
    n2 = jnp.minimum(t, u)
    work = s0
    mx = jnp.max(work, axis=-1, keepdims=True)
    m = mx
    for _t in range(_KK - 1):
        eq = work == mx
        work = jnp.where(eq, n1, work)
        n1 = jnp.where(eq, n2, n1)
        n2 = jnp.where(eq, n3, n2)
        n3 = jnp.where(eq, -3e38, n3)
        mx = jnp.max(work, axis=-1, keepdims=True)
    e = jnp.exp(logits - m)
    zs = jnp.sum(e, axis=-1, keepdims=True)
    sel = logits >= mx
    esel = jnp.where(sel, e, 0.0)
    denom = jnp.sum(esel, axis=-1, keepdims=True) + zs * 1e-9
    wfull = esel / denom
    agg = _mm(wfull, xd)
    xpad = jnp.concatenate(
        [xd[3 * 128:], jnp.zeros((4 * 128 - _NPG, _D), jnp.float32)], axis=0)
    seg_p = jnp.zeros((_RT * 8, _D), jnp.float32)
    deg_p = jnp.zeros((_RT * 8, 1), jnp.float32)
    for cg in range(_CT):
        acg = a_blk[:, cg].reshape(_RT * 8, 128)
        xr = xd[cg * 128:(cg + 1) * 128] if cg < _CT - 1 else xpad
        seg_p = seg_p + _mm(acg, xr)
        deg_p = deg_p + jnp.sum(acg, axis=-1, keepdims=True)
    seg = seg_p[:_NPG] / jnp.maximum(deg_p[:_NPG], 1.0)
    pre = _mm(seg, wg[:_D]) + _mm(agg, wg[_D:]) + bg
    return xu, xd + jnp.maximum(pre, 0.0)


def _layer_mid_body(z_ref, xd_ref, ssum_ref, ssq_ref, a_ref, bns_ref,
                    bnsh_ref, wq_ref, wk_ref, wg_ref, bg_ref, w1_ref, b1_ref,
                    w2_ref, b2_ref, xdo_ref, zo_ref, ssumo_ref, ssqo_ref):
    xu, xdn = _attn_update(z_ref[0], xd_ref[0], a_ref[0], ssum_ref[...],
                           ssq_ref[...], bns_ref[...], bnsh_ref[...],
                           wq_ref[...], wk_ref[...], wg_ref[...], bg_ref[...])
    xdo_ref[0] = xdn
    z2 = xu + _mm(jnp.maximum(_mm(xu, w1_ref[...]) + b1_ref[...], 0.0),
                  w2_ref[...]) + b2_ref[...]
    zo_ref[0] = z2
    ssumo_ref[0] = jnp.sum(z2, axis=0, keepdims=True)
    ssqo_ref[0] = jnp.sum(z2 * z2, axis=0, keepdims=True)


def _layer_last_body(z_ref, xd_ref, ssum_ref, ssq_ref, a_ref, bns_ref,
                     bnsh_ref, wq_ref, wk_ref, wg_ref, bg_ref, pool_ref):
    _xu, xdn = _attn_update(z_ref[0], xd_ref[0], a_ref[0], ssum_ref[...],
                            ssq_ref[...], bns_ref[...], bnsh_ref[...],
                            wq_ref[...], wk_ref[...], wg_ref[...], bg_ref[...])
    pool_ref[0] = jnp.sum(xdn, axis=0, keepdims=True) * (1.0 / _NPG)


def _head_body(p_ref, wm1_ref, bm1_ref, wm2_ref, bm2_ref, o_ref):
    hmid = jnp.maximum(_mm(p_ref[...], wm1_ref[...]) + bm1_ref[...], 0.0)
    o_ref[...] = _mm(hmid, wm2_ref[...]) + bm2_ref[...]


def _blk(shape, index_map):
    return pl.BlockSpec(shape, index_map)


def _full(shape):
    return pl.BlockSpec(shape, lambda *b: tuple(0 for _ in shape))


def _row(d2):
    return pl.BlockSpec((1, 1, d2), lambda b: (b, 0, 0))


def _node_block():
    return pl.BlockSpec((1, _NPG, _D), lambda b: (b, 0, 0))


def kernel(x, edge_index, batch, W_enc, b_enc, W1, b1, W2, b2, bn_scale,
           bn_shift, Wq, Wk, W_gnn, b_gnn, Wm1, bm1, Wm2, bm2):
    a3 = _build_adjacency(edge_index).reshape(_B, _RT, _CT, 8, 128)

    f32 = jnp.float32
    nd = jax.ShapeDtypeStruct((_B, _NPG, _D), f32)
    bd = jax.ShapeDtypeStruct((_B, 1, _D), f32)

    h, z, ssum, ssq = pl.pallas_call(
        _enc_body,
        grid=(_B,),
        in_specs=[_node_block(), _full((_D, _D)), _full((1, _D)),
                  _full((_D, _FF)), _full((1, _FF)), _full((_FF, _D)),
                  _full((1, _D))],
        out_specs=[_node_block(), _node_block(), _row(_D), _row(_D)],
        out_shape=[nd, nd, bd, bd],
    )(x.reshape(_B, _NPG, _D), W_enc, b_enc.reshape(1, _D), W1[0],
      b1[0].reshape(1, _FF), W2[0], b2[0].reshape(1, _D))

    xd = h
    layer_common_specs = [
        _node_block(), _node_block(), _full((_B, 1, _D)), _full((_B, 1, _D)),
        pl.BlockSpec((1, _RT, _CT, 8, 128), lambda b: (b, 0, 0, 0, 0)),
        _full((1, _D)), _full((1, _D)), _full((_D, _D)), _full((_D, _D)),
        _full((2 * _D, _D)), _full((1, _D)),
    ]
    for i in range(_L):
        common_args = (z, xd, ssum, ssq, a3, bn_scale[i].reshape(1, _D),
                       bn_shift[i].reshape(1, _D), Wq[i], Wk[i], W_gnn[i],
                       b_gnn[i].reshape(1, _D))
        if i < _L - 1:
            xd, z, ssum, ssq = pl.pallas_call(
                _layer_mid_body,
                grid=(_B,),
                in_specs=layer_common_specs + [
                    _full((_D, _FF)), _full((1, _FF)), _full((_FF, _D)),
                    _full((1, _D))],
                out_specs=[_node_block(), _node_block(), _row(_D), _row(_D)],
                out_shape=[nd, nd, bd, bd],
            )(*common_args, W1[i + 1], b1[i + 1].reshape(1, _FF), W2[i + 1],
              b2[i + 1].reshape(1, _D))
        else:
            pooled = pl.pallas_call(
                _layer_last_body,
                grid=(_B,),
                in_specs=layer_common_specs,
                out_specs=[_row(_D)],
                out_shape=[bd],
            )(*common_args)[0]

    out = pl.pallas_call(
        _head_body,
        in_specs=[_full((_B, _D)), _full((_D, _D)), _full((1, _D)),
                  _full((_D, _C)), _full((1, _C))],
        out_specs=_full((_B, _C)),
        out_shape=jax.ShapeDtypeStruct((_B, _C), f32),
    )(pooled.reshape(_B, _D), Wm1, bm1.reshape(1, _D), Wm2,
      bm2.reshape(1, _C))
    return out


# speedup vs baseline: 1.0807x; 1.0051x over previous
import jax
import jax.numpy as jnp
import numpy as np
from jax import lax
from jax.experimental import pallas as pl
from jax.experimental.pallas import tpu as pltpu
from jax.experimental.pallas import tpu_sc as plsc

_B = 20
_NPG = 500
_N = 10000
_E = 320000
_D = 128
_FF = 256
_L = 3
_KK = 16
_C = 10

_NC = 2
_NS = 16
_EPG = _E // _B
_GPC = _B // _NC
_WAVES = 5
_GPW = _GPC // _WAVES
_RT = (_NPG + 7) // 8
_CT = (_NPG + 127) // 128
_TCELLS = _RT * _CT * 1024
_WAVE_CELLS = _GPW * _TCELLS
_WAVE_EDGES = _GPW * _EPG
_EPT = _WAVE_EDGES // _NS
_CH = 128
_NCHUNK = (_EPT + _CH - 1) // _CH
_EBUF = _NCHUNK * _CH
_SLICE = _WAVE_CELLS // _NS
_SPMEM = _WAVE_CELLS + 8
_DUMP = _WAVE_CELLS


def _adj_body(src_hbm, dst_hbm, a_hbm, zero_v, stage_v, src_v, dst_v, idx_v,
              ones_v, shared, sem_out):
    cid = lax.axis_index("c")
    sid = lax.axis_index("s")

    zf = jnp.zeros((16,), jnp.float32)
    of = jnp.ones((16,), jnp.float32)

    def fill_body(j, _):
        zero_v[pl.ds(j * 16, 16)] = zf
        return 0

    lax.fori_loop(0, _SLICE // 16, fill_body, 0)
    for j in range(_CH // 16):
        ones_v[pl.ds(j * 16, 16)] = of

    out_cp = None
    for w in range(_WAVES):
        g0 = cid * _GPC + w * _GPW
        pltpu.sync_copy(zero_v, shared.at[pl.ds(sid * _SLICE, _SLICE)])
        ebase = g0 * _EPG + sid * _EPT
        pltpu.sync_copy(src_hbm.at[pl.ds(ebase, _EPT)],
                        src_v.at[pl.ds(0, _EPT)])
        pltpu.sync_copy(dst_hbm.at[pl.ds(ebase, _EPT)],
                        dst_v.at[pl.ds(0, _EPT)])
        plsc.subcore_barrier()
        lanes = lax.broadcasted_iota(jnp.int32, (16,), 0)

        def chunk_body(c, _):
            for j in range(_CH // 16):
                off = c * _CH + j * 16
                s = src_v[pl.ds(off, 16)]
                dv = dst_v[pl.ds(off, 16)]
                g = lax.shift_right_logical(dv * 16778, 23)
                dl = dv - g * _NPG
                sl = s - g * _NPG
                widx = ((g - g0) * _TCELLS
                        + (lax.shift_right_logical(dl, 3) * _CT
                           + lax.shift_right_logical(sl, 7)) * 1024
                        + (dl & 7) * 128 + (sl & 127))
                valid = (off + lanes) < _EPT
                idx_v[pl.ds(j * 16, 16)] = jnp.where(valid, widx, _DUMP)
            pltpu.sync_copy(ones_v, shared.at[idx_v], add=True)
            return 0

        lax.fori_loop(0, _NCHUNK, chunk_body, 0)
        plsc.subcore_barrier()
        if out_cp is not None:
            out_cp.wait()
        awave = g0 * _TCELLS
        pltpu.sync_copy(shared.at[pl.ds(sid * _SLICE, _SLICE)], stage_v)
        plsc.subcore_barrier()
        out_cp = pltpu.async_copy(
            stage_v, a_hbm.at[pl.ds(awave + sid * _SLICE, _SLICE)], sem_out)
    out_cp.wait()


def _build_adjacency(edge_index):
    mesh = plsc.VectorSubcoreMesh(core_axis_name="c", subcore_axis_name="s",
                                  num_cores=_NC, num_subcores=_NS)
    fn = pl.kernel(
        _adj_body,
        out_type=jax.ShapeDtypeStruct((_B * _TCELLS,), jnp.float32),
        mesh=mesh,
        scratch_types=[
            pltpu.VMEM((_SLICE,), jnp.float32),
            pltpu.VMEM((_SLICE,), jnp.float32),
            pltpu.VMEM((_EBUF,), jnp.int32),
            pltpu.VMEM((_EBUF,), jnp.int32),
            pltpu.VMEM((_CH,), jnp.int32),
            pltpu.VMEM((_CH,), jnp.float32),
            pltpu.VMEM_SHARED((_SPMEM,), jnp.float32),
            pltpu.SemaphoreType.DMA,
        ],
    )
    return fn(edge_index[0], edge_index[1])


def _mm(a, b):
    return jnp.dot(a, b, preferred_element_type=jnp.float32)


def _enc_body(x_ref, wenc_ref, benc_ref, w1_ref, b1_ref, w2_ref, b2_ref,
              h_ref, z_ref, ssum_ref, ssq_ref):
    xg = x_ref[0]
    h = _mm(xg, wenc_ref[...]) + benc_ref[...]
    z = h + _mm(jnp.maximum(_mm(h, w1_ref[...]) + b1_ref[...], 0.0),
                w2_ref[...]) + b2_ref[...]
    h_ref[0] = h
    z_ref[0] = z
    ssum_ref[0] = jnp.sum(z, axis=0, keepdims=True)
    ssq_ref[0] = jnp.sum(z * z, axis=0, keepdims=True)


def _attn_update(z, xd, a_blk, ssum, ssq, bns, bnsh, wq, wk, wg, bg):
    mu = jnp.sum(ssum, axis=0) * (1.0 / _N)
    ex2 = jnp.sum(ssq, axis=0) * (1.0 / _N)
    var = ex2 - mu * mu
    xu = (z - mu) / jnp.sqrt(var + 1e-5) * bns + bnsh
    q = _mm(xu, wq)
    k = _mm(xu, wk)
    logits = lax.dot_general(q, k, (((1,), (1,)), ((), ())),
                             preferred_element_type=jnp.float32)
    logits = logits * (1.0 / np.sqrt(_D))
    neg = jnp.full((_NPG, 12), -3e38, jnp.float32)
    x0 = logits[:, 0:128]
    x1 = logits[:, 128:256]
    x2 = logits[:, 256:384]
    x3 = jnp.concatenate([logits[:, 384:500], neg], axis=1)
    a = jnp.maximum(x0, x1)
    b = jnp.minimum(x0, x1)
    c = jnp.maximum(x2, x3)
    d = jnp.minimum(x2, x3)
    s0 = jnp.maximum(a, c)
    t = jnp.minimum(a, c)
    u = jnp.maximum(b, d)
    n3 = jnp.minimum(b, d)
    n1 = jnp.maximum(t, u)
    n2 = jnp.minimum(t, u)
    work = s0
    mx = jnp.max(work, axis=-1, keepdims=True)
    m = mx
    for _t in range(_KK - 1):
        eq = work == mx
        work = jnp.where(eq, n1, work)
        n1 = jnp.where(eq, n2, n1)
        n2 = jnp.where(eq, n3, n2)
        n3 = jnp.where(eq, -3e38, n3)
        mx = jnp.max(work, axis=-1, keepdims=True)
    e = jnp.exp(logits - m)
    zs = jnp.sum(e, axis=-1, keepdims=True)
    sel = logits >= mx
    esel = jnp.where(sel, e, 0.0)
    denom = jnp.sum(esel, axis=-1, keepdims=True) + zs * 1e-9
    wfull = esel / denom
    agg = _mm(wfull, xd)
    xpad = jnp.concatenate(
        [xd[3 * 128:], jnp.zeros((4 * 128 - _NPG, _D), jnp.float32)], axis=0)
    seg_p = jnp.zeros((_RT * 8, _D), jnp.float32)
    deg_p = jnp.zeros((_RT * 8, 1), jnp.float32)
    for cg in range(_CT):
        acg = a_blk[:, cg].reshape(_RT * 8, 128)
        xr = xd[cg * 128:(cg + 1) * 128] if cg < _CT - 1 else xpad
        seg_p = seg_p + _mm(acg, xr)
        deg_p = deg_p + jnp.sum(acg, axis=-1, keepdims=True)
    seg = seg_p[:_NPG] / jnp.maximum(deg_p[:_NPG], 1.0)
    pre = _mm(seg, wg[:_D]) + _mm(agg, wg[_D:]) + bg
    return xu, xd + jnp.maximum(pre, 0.0)


def _layer_mid_body(z_ref, xd_ref, ssum_ref, ssq_ref, a_ref, bns_ref,
                    bnsh_ref, wq_ref, wk_ref, wg_ref, bg_ref, w1_ref, b1_ref,
                    w2_ref, b2_ref, xdo_ref, zo_ref, ssumo_ref, ssqo_ref):
    xu, xdn = _attn_update(z_ref[0], xd_ref[0], a_ref[0], ssum_ref[...],
                           ssq_ref[...], bns_ref[...], bnsh_ref[...],
                           wq_ref[...], wk_ref[...], wg_ref[...], bg_ref[...])
    xdo_ref[0] = xdn
    z2 = xu + _mm(jnp.maximum(_mm(xu, w1_ref[...]) + b1_ref[...], 0.0),
                  w2_ref[...]) + b2_ref[...]
    zo_ref[0] = z2
    ssumo_ref[0] = jnp.sum(z2, axis=0, keepdims=True)
    ssqo_ref[0] = jnp.sum(z2 * z2, axis=0, keepdims=True)


def _layer_last_body(z_ref, xd_ref, ssum_ref, ssq_ref, a_ref, bns_ref,
                     bnsh_ref, wq_ref, wk_ref, wg_ref, bg_ref, pool_ref):
    _xu, xdn = _attn_update(z_ref[0], xd_ref[0], a_ref[0], ssum_ref[...],
                            ssq_ref[...], bns_ref[...], bnsh_ref[...],
                            wq_ref[...], wk_ref[...], wg_ref[...], bg_ref[...])
    pool_ref[0] = jnp.sum(xdn, axis=0, keepdims=True) * (1.0 / _NPG)


def _head_body(p_ref, wm1_ref, bm1_ref, wm2_ref, bm2_ref, o_ref):
    hmid = jnp.maximum(_mm(p_ref[...], wm1_ref[...]) + bm1_ref[...], 0.0)
    o_ref[...] = _mm(hmid, wm2_ref[...]) + bm2_ref[...]


def _blk(shape, index_map):
    return pl.BlockSpec(shape, index_map)


def _full(shape):
    return pl.BlockSpec(shape, lambda *b: tuple(0 for _ in shape))


def _row(d2):
    return pl.BlockSpec((1, 1, d2), lambda b: (b, 0, 0))


def _node_block():
    return pl.BlockSpec((1, _NPG, _D), lambda b: (b, 0, 0))


def kernel(x, edge_index, batch, W_enc, b_enc, W1, b1, W2, b2, bn_scale,
           bn_shift, Wq, Wk, W_gnn, b_gnn, Wm1, bm1, Wm2, bm2):
    a3 = _build_adjacency(edge_index).reshape(_B, _RT, _CT, 8, 128)

    f32 = jnp.float32
    nd = jax.ShapeDtypeStruct((_B, _NPG, _D), f32)
    bd = jax.ShapeDtypeStruct((_B, 1, _D), f32)

    h, z, ssum, ssq = pl.pallas_call(
        _enc_body,
        grid=(_B,),
        in_specs=[_node_block(), _full((_D, _D)), _full((1, _D)),
                  _full((_D, _FF)), _full((1, _FF)), _full((_FF, _D)),
                  _full((1, _D))],
        out_specs=[_node_block(), _node_block(), _row(_D), _row(_D)],
        out_shape=[nd, nd, bd, bd],
    )(x.reshape(_B, _NPG, _D), W_enc, b_enc.reshape(1, _D), W1[0],
      b1[0].reshape(1, _FF), W2[0], b2[0].reshape(1, _D))

    xd = h
    layer_common_specs = [
        _node_block(), _node_block(), _full((_B, 1, _D)), _full((_B, 1, _D)),
        pl.BlockSpec((1, _RT, _CT, 8, 128), lambda b: (b, 0, 0, 0, 0)),
        _full((1, _D)), _full((1, _D)), _full((_D, _D)), _full((_D, _D)),
        _full((2 * _D, _D)), _full((1, _D)),
    ]
    for i in range(_L):
        common_args = (z, xd, ssum, ssq, a3, bn_scale[i].reshape(1, _D),
                       bn_shift[i].reshape(1, _D), Wq[i], Wk[i], W_gnn[i],
                       b_gnn[i].reshape(1, _D))
        if i < _L - 1:
            xd, z, ssum, ssq = pl.pallas_call(
                _layer_mid_body,
                grid=(_B,),
                in_specs=layer_common_specs + [
                    _full((_D, _FF)), _full((1, _FF)), _full((_FF, _D)),
                    _full((1, _D))],
                out_specs=[_node_block(), _node_block(), _row(_D), _row(_D)],
                out_shape=[nd, nd, bd, bd],
            )(*common_args, W1[i + 1], b1[i + 1].reshape(1, _FF), W2[i + 1],
              b2[i + 1].reshape(1, _D))
        else:
            pooled = pl.pallas_call(
                _layer_last_body,
                grid=(_B,),
                in_specs=layer_common_specs,
                out_specs=[_row(_D)],
                out_shape=[bd],
            )(*common_args)[0]

    out = pl.pallas_call(
        _head_body,
        in_specs=[_full((_B, _D)), _full((_D, _D)), _full((1, _D)),
                  _full((_D, _C)), _full((1, _C))],
        out_specs=_full((_B, _C)),
        out_shape=jax.ShapeDtypeStruct((_B, _C), f32),
    )(pooled.reshape(_B, _D), Wm1, bm1.reshape(1, _D), Wm2,
      bm2.reshape(1, _C))
    return out
